# Initial kernel scaffold; baseline (speedup 1.0000x reference)
import jax
import jax.numpy as jnp
from jax.experimental import pallas as pl


def kernel(node_features, adj_data, adj_row, adj_col, W_in, b_in, ln_in_scale, ln_in_offset, W_gc, b_gc, ln_fp_scale, ln_fp_offset, W_out, b_out):
    raise NotImplementedError("write your pallas kernel here")


# speedup vs baseline: 1.1098x; 1.1098x over previous
import functools

import jax
import jax.numpy as jnp
from jax import lax
from jax.experimental import pallas as pl
from jax.experimental.pallas import tpu as pltpu

N = 10000
D = 128
H = 128
MAXITER = 16
BLK = 1000


def _ln(x, scale, offset, eps=1e-5):
    mu = jnp.mean(x, axis=-1, keepdims=True)
    var = jnp.mean((x - mu) ** 2, axis=-1, keepdims=True)
    return (x - mu) * lax.rsqrt(var + eps) * scale + offset


def _input_body(nf, W_in, b_in, lns, lno, bgc, fps, fpo, W_gc,
                x_ref, z_ref, h_ref):
    x = jnp.dot(nf[...], W_in[...], preferred_element_type=jnp.float32)
    x = jnp.maximum(x + b_in[...], 0.0)
    x = _ln(x, lns[...], lno[...])
    x_ref[...] = x
    z = _ln(jnp.maximum(bgc[...] + x, 0.0), fps[...], fpo[...])
    z_ref[...] = z
    h_ref[...] = jnp.dot(z, W_gc[...], preferred_element_type=jnp.float32)


def _epi_body(agg, x, bgc, fps, fpo, W_gc, z_ref, h_ref):
    a = agg[...] + bgc[...]
    z = _ln(jnp.maximum(a + x[...], 0.0), fps[...], fpo[...])
    z_ref[...] = z
    h_ref[...] = jnp.dot(z, W_gc[...], preferred_element_type=jnp.float32)


def _out_body(z, W_out, b_out, o_ref):
    o_ref[...] = jnp.dot(z[...], W_out[...],
                         preferred_element_type=jnp.float32) + b_out[...]


def _row_spec():
    return pl.BlockSpec((BLK, H), lambda i: (i, 0))


def _full_spec(r, c):
    return pl.BlockSpec((r, c), lambda i: (0, 0))


def _vec_spec(c):
    return pl.BlockSpec((c,), lambda i: (0,))


def kernel(node_features, adj_data, adj_row, adj_col, W_in, b_in,
           ln_in_scale, ln_in_offset, W_gc, b_gc, ln_fp_scale, ln_fp_offset,
           W_out, b_out):
    n = node_features.shape[0]
    grid = (n // BLK,)

    x, z, h = pl.pallas_call(
        _input_body,
        grid=grid,
        in_specs=[_row_spec(), _full_spec(D, H), _vec_spec(H), _vec_spec(H),
                  _vec_spec(H), _vec_spec(H), _vec_spec(H), _vec_spec(H),
                  _full_spec(H, H)],
        out_specs=[_row_spec(), _row_spec(), _row_spec()],
        out_shape=[jax.ShapeDtypeStruct((n, H), jnp.float32)] * 3,
    )(node_features, W_in, b_in, ln_in_scale, ln_in_offset, b_gc,
      ln_fp_scale, ln_fp_offset, W_gc)

    epi = pl.pallas_call(
        _epi_body,
        grid=grid,
        in_specs=[_row_spec(), _row_spec(), _vec_spec(H), _vec_spec(H),
                  _vec_spec(H), _full_spec(H, H)],
        out_specs=[_row_spec(), _row_spec()],
        out_shape=[jax.ShapeDtypeStruct((n, H), jnp.float32)] * 2,
    )

    def body(carry, _):
        z, h = carry
        msg = adj_data[:, None] * jnp.take(h, adj_col, axis=0)
        agg = jax.ops.segment_sum(msg, adj_row, num_segments=n)
        z, h = epi(agg, x, b_gc, ln_fp_scale, ln_fp_offset, W_gc)
        return (z, h), None

    (z, h), _ = lax.scan(body, (z, h), None, length=MAXITER - 1)

    nc = W_out.shape[1]
    preds = pl.pallas_call(
        _out_body,
        grid=grid,
        in_specs=[_row_spec(), _full_spec(H, nc), _vec_spec(nc)],
        out_specs=pl.BlockSpec((BLK, nc), lambda i: (i, 0)),
        out_shape=jax.ShapeDtypeStruct((n, nc), jnp.float32),
    )(z, W_out, b_out)
    return preds


# speedup vs baseline: 3.4994x; 3.1530x over previous
import functools

import jax
import jax.numpy as jnp
from jax import lax
from jax.experimental import pallas as pl
from jax.experimental.pallas import tpu as pltpu
from jax.experimental.pallas import tpu_sc as plsc

H = 128
MAXITER = 16
BLK = 1000
NC = 2
NS = 16
NW = NC * NS
L = 16
SUB = 128
KSUB = 2
C = SUB * KSUB


def _ln(x, scale, offset, eps=1e-5):
    mu = jnp.mean(x, axis=-1, keepdims=True)
    var = jnp.mean((x - mu) ** 2, axis=-1, keepdims=True)
    return (x - mu) * lax.rsqrt(var + eps) * scale + offset


def _input_body(nf, W_in, b_in, lns, lno, bgc, fps, fpo, W_gc,
                x_ref, z_ref, h_ref):
    x = jnp.dot(nf[...], W_in[...], preferred_element_type=jnp.float32)
    x = jnp.maximum(x + b_in[...], 0.0)
    x = _ln(x, lns[...], lno[...])
    x_ref[...] = x
    z = _ln(jnp.maximum(bgc[...] + x, 0.0), fps[...], fpo[...])
    z_ref[...] = z
    h_ref[...] = jnp.dot(z, W_gc[...], preferred_element_type=jnp.float32)


def _epi_body(aggA, aggB, x, bgc, fps, fpo, W_gc, z_ref, h_ref):
    a = aggA[...] + aggB[...] + bgc[...]
    z = _ln(jnp.maximum(a + x[...], 0.0), fps[...], fpo[...])
    z_ref[...] = z
    h_ref[...] = jnp.dot(z, W_gc[...], preferred_element_type=jnp.float32)


def _out_body(z, W_out, b_out, o_ref):
    o_ref[...] = jnp.dot(z[...], W_out[...],
                         preferred_element_type=jnp.float32) + b_out[...]


def _row_spec():
    return pl.BlockSpec((BLK, H), lambda i: (i, 0))


def _full_spec(r, c):
    return pl.BlockSpec((r, c), lambda i: (0, 0))


def _vec_spec(c):
    return pl.BlockSpec((c,), lambda i: (0,))


def _make_sc_agg(n, rows_per_tile):
    assert rows_per_tile % KSUB == 0
    nchunks = rows_per_tile // KSUB
    zr = -(-n // NS)
    zr = -(-zr // 8) * 8
    np_pad = zr * NS

    mesh = plsc.VectorSubcoreMesh(core_axis_name="c", subcore_axis_name="s")

    @functools.partial(
        pl.kernel,
        out_type=jax.ShapeDtypeStruct((NC, np_pad, H), jnp.float32),
        mesh=mesh,
        scratch_types=[
            pltpu.VMEM((KSUB, SUB), jnp.int32),
            pltpu.VMEM((KSUB, SUB), jnp.int32),
            pltpu.VMEM((KSUB, SUB), jnp.float32),
            pltpu.VMEM((C, H), jnp.float32),
            pltpu.VMEM_SHARED((np_pad, H), jnp.float32),
            pltpu.SemaphoreType.DMA,
        ],
    )
    def sc_agg(h_hbm, col_hbm, row_hbm, data_hbm, out_hbm,
               col_v, row_v, data_v, msg_v, agg_sh, sem):
        cid = lax.axis_index("c")
        sid = lax.axis_index("s")
        wid = cid * NS + sid

        def zrow(r, _):
            z16 = jnp.zeros((L,), jnp.float32)
            for j in range(H // L):
                msg_v[r, pl.ds(j * L, L)] = z16
            return _
        lax.fori_loop(0, C, zrow, None)
        base_z = sid * zr
        left = zr
        off = 0
        while left > 0:
            step = min(left, C)
            pltpu.sync_copy(msg_v.at[pl.ds(0, step)],
                            agg_sh.at[pl.ds(base_z + off, step)])
            off += step
            left -= step
        plsc.subcore_barrier()

        def chunk(k, _):
            roff = wid * rows_per_tile + k * KSUB
            pltpu.sync_copy(col_hbm.at[pl.ds(roff, KSUB)], col_v)
            pltpu.sync_copy(row_hbm.at[pl.ds(roff, KSUB)], row_v)
            pltpu.sync_copy(data_hbm.at[pl.ds(roff, KSUB)], data_v)
            cps = [
                pltpu.async_copy(h_hbm.at[col_v.at[j]],
                                 msg_v.at[pl.ds(j * SUB, SUB)], sem)
                for j in range(KSUB)
            ]
            for cp in cps:
                cp.wait()

            def group(g, _):
                d16 = data_v[g // (SUB // L),
                             pl.ds((g % (SUB // L)) * L, L)]
                for e in range(L):
                    d = d16[e]
                    r = g * L + e
                    for j in range(H // L):
                        sl = pl.ds(j * L, L)
                        msg_v[r, sl] = msg_v[r, sl] * d
                return _
            lax.fori_loop(0, C // L, group, None)

            for j in range(KSUB):
                pltpu.sync_copy(msg_v.at[pl.ds(j * SUB, SUB)],
                                agg_sh.at[row_v.at[j]], add=True)
            return _
        lax.fori_loop(0, nchunks, chunk, None)
        plsc.subcore_barrier()

        pltpu.sync_copy(agg_sh.at[pl.ds(sid * zr, zr)],
                        out_hbm.at[cid, pl.ds(sid * zr, zr)])

    return sc_agg, np_pad


def kernel(node_features, adj_data, adj_row, adj_col, W_in, b_in,
           ln_in_scale, ln_in_offset, W_gc, b_gc, ln_fp_scale, ln_fp_offset,
           W_out, b_out):
    n = node_features.shape[0]
    e_total = adj_data.shape[0]
    grid = (n // BLK,)

    ew = -(-e_total // (NW * C)) * C
    e_pad = NW * ew - e_total
    rows_per_tile = ew // SUB

    def pad2(a):
        return jnp.concatenate(
            [a, jnp.zeros((e_pad,), a.dtype)]).reshape(-1, SUB)

    col2 = pad2(adj_col)
    row2 = pad2(adj_row)
    data2 = pad2(adj_data)

    x, z, h = pl.pallas_call(
        _input_body,
        grid=grid,
        in_specs=[_row_spec(), _full_spec(H, H), _vec_spec(H), _vec_spec(H),
                  _vec_spec(H), _vec_spec(H), _vec_spec(H), _vec_spec(H),
                  _full_spec(H, H)],
        out_specs=[_row_spec(), _row_spec(), _row_spec()],
        out_shape=[jax.ShapeDtypeStruct((n, H), jnp.float32)] * 3,
    )(node_features, W_in, b_in, ln_in_scale, ln_in_offset, b_gc,
      ln_fp_scale, ln_fp_offset, W_gc)

    sc_agg, np_pad = _make_sc_agg(n, rows_per_tile)

    epi = pl.pallas_call(
        _epi_body,
        grid=grid,
        in_specs=[_row_spec(), _row_spec(), _row_spec(), _vec_spec(H),
                  _vec_spec(H), _vec_spec(H), _full_spec(H, H)],
        out_specs=[_row_spec(), _row_spec()],
        out_shape=[jax.ShapeDtypeStruct((n, H), jnp.float32)] * 2,
    )

    def body(carry, _):
        z, h = carry
        agg = sc_agg(h, col2, row2, data2)
        z, h = epi(agg[0], agg[1], x, b_gc, ln_fp_scale, ln_fp_offset, W_gc)
        return (z, h), None

    (z, h), _ = lax.scan(body, (z, h), None, length=MAXITER - 1)

    nc_out = W_out.shape[1]
    preds = pl.pallas_call(
        _out_body,
        grid=grid,
        in_specs=[_row_spec(), _full_spec(H, nc_out), _vec_spec(nc_out)],
        out_specs=pl.BlockSpec((BLK, nc_out), lambda i: (i, 0)),
        out_shape=jax.ShapeDtypeStruct((n, nc_out), jnp.float32),
    )(z, W_out, b_out)
    return preds


# speedup vs baseline: 4.4138x; 1.2613x over previous
import functools

import jax
import jax.numpy as jnp
from jax import lax
from jax.experimental import pallas as pl
from jax.experimental.pallas import tpu as pltpu
from jax.experimental.pallas import tpu_sc as plsc

H = 128
MAXITER = 16
BLK = 1000
NC = 2
NS = 16
NW = NC * NS
L = 16
C = 128
RING_I = 4
RING_M = 2


def _ln(x, scale, offset, eps=1e-5):
    mu = jnp.mean(x, axis=-1, keepdims=True)
    var = jnp.mean((x - mu) ** 2, axis=-1, keepdims=True)
    return (x - mu) * lax.rsqrt(var + eps) * scale + offset


def _input_body(nf, W_in, b_in, lns, lno, bgc, fps, fpo, W_gc,
                x_ref, z_ref, h_ref):
    x = jnp.dot(nf[...], W_in[...], preferred_element_type=jnp.float32)
    x = jnp.maximum(x + b_in[...], 0.0)
    x = _ln(x, lns[...], lno[...])
    x_ref[...] = x
    z = _ln(jnp.maximum(bgc[...] + x, 0.0), fps[...], fpo[...])
    z_ref[...] = z
    h_ref[...] = jnp.dot(z, W_gc[...], preferred_element_type=jnp.float32)


def _epi_body(aggA, aggB, x, bgc, fps, fpo, W_gc, z_ref, h_ref):
    a = aggA[...] + aggB[...] + bgc[...]
    z = _ln(jnp.maximum(a + x[...], 0.0), fps[...], fpo[...])
    z_ref[...] = z
    h_ref[...] = jnp.dot(z, W_gc[...], preferred_element_type=jnp.float32)


def _out_body(z, W_out, b_out, o_ref):
    o_ref[...] = jnp.dot(z[...], W_out[...],
                         preferred_element_type=jnp.float32) + b_out[...]


def _row_spec():
    return pl.BlockSpec((BLK, H), lambda i: (i, 0))


def _full_spec(r, c):
    return pl.BlockSpec((r, c), lambda i: (0, 0))


def _vec_spec(c):
    return pl.BlockSpec((c,), lambda i: (0,))


def _make_sc_agg(n, rows_per_tile):
    assert rows_per_tile % RING_I == 0
    nchunks = rows_per_tile
    zr = -(-n // NS)
    zr = -(-zr // C) * C
    np_pad = zr * NS

    mesh = plsc.VectorSubcoreMesh(core_axis_name="c", subcore_axis_name="s")

    @functools.partial(
        pl.kernel,
        out_type=jax.ShapeDtypeStruct((NC, np_pad, H), jnp.float32),
        mesh=mesh,
        scratch_types=[
            pltpu.VMEM((RING_I, C), jnp.int32),
            pltpu.VMEM((RING_I, C), jnp.int32),
            pltpu.VMEM((RING_I, C), jnp.float32),
            pltpu.VMEM((RING_M, C, H), jnp.float32),
            pltpu.VMEM_SHARED((np_pad, H), jnp.float32),
            [pltpu.SemaphoreType.DMA] * RING_I,
            [pltpu.SemaphoreType.DMA] * RING_M,
            [pltpu.SemaphoreType.DMA] * RING_M,
        ],
    )
    def sc_agg(h_hbm, col_hbm, row_hbm, data_hbm, out_hbm,
               col_v, row_v, data_v, msg_v, agg_sh, si, sg, ss):
        cid = lax.axis_index("c")
        sid = lax.axis_index("s")
        wid = cid * NS + sid
        tbase = wid * rows_per_tile

        def start_idx(k, slot):
            pltpu.async_copy(col_hbm.at[tbase + k], col_v.at[slot], si[slot])
            pltpu.async_copy(row_hbm.at[tbase + k], row_v.at[slot], si[slot])
            pltpu.async_copy(data_hbm.at[tbase + k], data_v.at[slot],
                             si[slot])

        def wait_idx(k, slot):
            pltpu.make_async_copy(col_hbm.at[tbase + k], col_v.at[slot],
                                  si[slot]).wait()
            pltpu.make_async_copy(row_hbm.at[tbase + k], row_v.at[slot],
                                  si[slot]).wait()
            pltpu.make_async_copy(data_hbm.at[tbase + k], data_v.at[slot],
                                  si[slot]).wait()

        def start_gather(slot, m):
            pltpu.async_copy(h_hbm.at[col_v.at[slot]], msg_v.at[m], sg[m])

        def wait_gather(slot, m):
            pltpu.make_async_copy(h_hbm.at[col_v.at[slot]], msg_v.at[m],
                                  sg[m]).wait()

        def start_scatter(slot, m):
            pltpu.async_copy(msg_v.at[m], agg_sh.at[row_v.at[slot]], ss[m],
                             add=True)

        def wait_scatter(slot, m):
            pltpu.make_async_copy(msg_v.at[m], agg_sh.at[row_v.at[slot]],
                                  ss[m]).wait()

        @plsc.parallel_loop(0, C)
        def _(r):
            z16 = jnp.zeros((L,), jnp.float32)
            for j in range(H // L):
                msg_v[0, r, pl.ds(j * L, L)] = z16
        for i in range(zr // C):
            pltpu.sync_copy(msg_v.at[0],
                            agg_sh.at[pl.ds(sid * zr + i * C, C)])
        plsc.subcore_barrier()

        start_idx(0, 0)
        start_idx(1, 1)
        wait_idx(0, 0)
        start_gather(0, 0)

        @pl.loop(0, nchunks, step=RING_I)
        def _(k0):
            for b in range(RING_I):
                kk = k0 + b
                m = b % 2
                bn = (b + 1) % RING_I
                bnn = (b + 2) % RING_I
                bp = (b + 3) % RING_I

                @pl.when(kk >= 1)
                def _():
                    wait_scatter(bp, 1 - m)

                @pl.when(kk < nchunks - 1)
                def _():
                    wait_idx(kk + 1, bn)
                    start_gather(bn, 1 - m)

                @pl.when(kk < nchunks - 2)
                def _():
                    start_idx(kk + 2, bnn)

                wait_gather(b, m)

                @plsc.parallel_loop(0, C // L, unroll=2)
                def _(g):
                    d16 = data_v[b, pl.ds(g * L, L)]
                    for e in range(L):
                        d = d16[e]
                        r = g * L + e
                        for j in range(H // L):
                            sl = pl.ds(j * L, L)
                            msg_v[m, r, sl] = msg_v[m, r, sl] * d

                start_scatter(b, m)

        wait_scatter((nchunks - 1) % RING_I, (nchunks - 1) % 2)
        plsc.subcore_barrier()

        pltpu.sync_copy(agg_sh.at[pl.ds(sid * zr, zr)],
                        out_hbm.at[cid, pl.ds(sid * zr, zr)])

    return sc_agg, np_pad


def kernel(node_features, adj_data, adj_row, adj_col, W_in, b_in,
           ln_in_scale, ln_in_offset, W_gc, b_gc, ln_fp_scale, ln_fp_offset,
           W_out, b_out):
    n = node_features.shape[0]
    e_total = adj_data.shape[0]
    grid = (n // BLK,)

    gran = RING_I * C
    ew = -(-e_total // (NW * gran)) * gran
    e_pad = NW * ew - e_total
    rows_per_tile = ew // C

    def pad2(a):
        return jnp.concatenate(
            [a, jnp.zeros((e_pad,), a.dtype)]).reshape(-1, C)

    col2 = pad2(adj_col)
    row2 = pad2(adj_row)
    data2 = pad2(adj_data)

    x, z, h = pl.pallas_call(
        _input_body,
        grid=grid,
        in_specs=[_row_spec(), _full_spec(H, H), _vec_spec(H), _vec_spec(H),
                  _vec_spec(H), _vec_spec(H), _vec_spec(H), _vec_spec(H),
                  _full_spec(H, H)],
        out_specs=[_row_spec(), _row_spec(), _row_spec()],
        out_shape=[jax.ShapeDtypeStruct((n, H), jnp.float32)] * 3,
    )(node_features, W_in, b_in, ln_in_scale, ln_in_offset, b_gc,
      ln_fp_scale, ln_fp_offset, W_gc)

    sc_agg, np_pad = _make_sc_agg(n, rows_per_tile)

    epi = pl.pallas_call(
        _epi_body,
        grid=grid,
        in_specs=[_row_spec(), _row_spec(), _row_spec(), _vec_spec(H),
                  _vec_spec(H), _vec_spec(H), _full_spec(H, H)],
        out_specs=[_row_spec(), _row_spec()],
        out_shape=[jax.ShapeDtypeStruct((n, H), jnp.float32)] * 2,
    )

    def body(carry, _):
        z, h = carry
        agg = sc_agg(h, col2, row2, data2)
        z, h = epi(agg[0], agg[1], x, b_gc, ln_fp_scale, ln_fp_offset, W_gc)
        return (z, h), None

    (z, h), _ = lax.scan(body, (z, h), None, length=MAXITER - 1)

    nc_out = W_out.shape[1]
    preds = pl.pallas_call(
        _out_body,
        grid=grid,
        in_specs=[_row_spec(), _full_spec(H, nc_out), _vec_spec(nc_out)],
        out_specs=pl.BlockSpec((BLK, nc_out), lambda i: (i, 0)),
        out_shape=jax.ShapeDtypeStruct((n, nc_out), jnp.float32),
    )(z, W_out, b_out)
    return preds


# speedup vs baseline: 4.4219x; 1.0018x over previous
import functools

import jax
import jax.numpy as jnp
from jax import lax
from jax.experimental import pallas as pl
from jax.experimental.pallas import tpu as pltpu
from jax.experimental.pallas import tpu_sc as plsc

H = 128
MAXITER = 16
BLK = 1000
NC = 2
NS = 16
NW = NC * NS
L = 16
C = 128
RING_I = 4
RING_M = 2


def _ln(x, scale, offset, eps=1e-5):
    mu = jnp.mean(x, axis=-1, keepdims=True)
    var = jnp.mean((x - mu) ** 2, axis=-1, keepdims=True)
    return (x - mu) * lax.rsqrt(var + eps) * scale + offset


def _input_body(nf, W_in, b_in, lns, lno, bgc, fps, fpo, W_gc,
                x_ref, z_ref, h_ref):
    x = jnp.dot(nf[...], W_in[...], preferred_element_type=jnp.float32)
    x = jnp.maximum(x + b_in[...], 0.0)
    x = _ln(x, lns[...], lno[...])
    x_ref[...] = x
    z = _ln(jnp.maximum(bgc[...] + x, 0.0), fps[...], fpo[...])
    z_ref[...] = z
    h_ref[...] = jnp.dot(z, W_gc[...], preferred_element_type=jnp.float32)


def _epi_body(aggA, aggB, x, bgc, fps, fpo, W_gc, z_ref, h_ref):
    a = aggA[...] + aggB[...] + bgc[...]
    z = _ln(jnp.maximum(a + x[...], 0.0), fps[...], fpo[...])
    z_ref[...] = z
    h_ref[...] = jnp.dot(z, W_gc[...], preferred_element_type=jnp.float32)


def _out_body(z, W_out, b_out, o_ref):
    o_ref[...] = jnp.dot(z[...], W_out[...],
                         preferred_element_type=jnp.float32) + b_out[...]


def _row_spec():
    return pl.BlockSpec((BLK, H), lambda i: (i, 0))


def _full_spec(r, c):
    return pl.BlockSpec((r, c), lambda i: (0, 0))


def _vec_spec(c):
    return pl.BlockSpec((c,), lambda i: (0,))


def _make_sc_agg(n, rows_per_tile):
    assert rows_per_tile % RING_I == 0
    nchunks = rows_per_tile
    zr = -(-n // NS)
    zr = -(-zr // C) * C
    np_pad = zr * NS

    mesh = plsc.VectorSubcoreMesh(core_axis_name="c", subcore_axis_name="s")

    @functools.partial(
        pl.kernel,
        out_type=jax.ShapeDtypeStruct((NC, np_pad, H), jnp.float32),
        mesh=mesh,
        scratch_types=[
            pltpu.VMEM((RING_I, C), jnp.int32),
            pltpu.VMEM((RING_I, C), jnp.int32),
            pltpu.VMEM((RING_I, C), jnp.float32),
            pltpu.VMEM((RING_M, C, H), jnp.float32),
            pltpu.VMEM_SHARED((np_pad, H), jnp.float32),
            [pltpu.SemaphoreType.DMA] * RING_I,
            [pltpu.SemaphoreType.DMA] * RING_M,
            [pltpu.SemaphoreType.DMA] * RING_M,
        ],
    )
    def sc_agg(h_hbm, col_hbm, row_hbm, data_hbm, out_hbm,
               col_v, row_v, data_v, msg_v, agg_sh, si, sg, ss):
        cid = lax.axis_index("c")
        sid = lax.axis_index("s")
        wid = cid * NS + sid
        tbase = wid * rows_per_tile

        def start_idx(k, slot):
            pltpu.async_copy(col_hbm.at[tbase + k], col_v.at[slot], si[slot])
            pltpu.async_copy(row_hbm.at[tbase + k], row_v.at[slot], si[slot])
            pltpu.async_copy(data_hbm.at[tbase + k], data_v.at[slot],
                             si[slot])

        def wait_idx(k, slot):
            pltpu.make_async_copy(col_hbm.at[tbase + k], col_v.at[slot],
                                  si[slot]).wait()
            pltpu.make_async_copy(row_hbm.at[tbase + k], row_v.at[slot],
                                  si[slot]).wait()
            pltpu.make_async_copy(data_hbm.at[tbase + k], data_v.at[slot],
                                  si[slot]).wait()

        def start_gather(slot, m):
            pltpu.async_copy(h_hbm.at[col_v.at[slot]], msg_v.at[m], sg[m])

        def wait_gather(slot, m):
            pltpu.make_async_copy(h_hbm.at[col_v.at[slot]], msg_v.at[m],
                                  sg[m]).wait()

        def start_scatter(slot, m):
            pltpu.async_copy(msg_v.at[m], agg_sh.at[row_v.at[slot]], ss[m],
                             add=True)

        def wait_scatter(slot, m):
            pltpu.make_async_copy(msg_v.at[m], agg_sh.at[row_v.at[slot]],
                                  ss[m]).wait()

        @plsc.parallel_loop(0, C)
        def _(r):
            z16 = jnp.zeros((L,), jnp.float32)
            for j in range(H // L):
                msg_v[1, r, pl.ds(j * L, L)] = z16
        start_idx(0, 0)
        start_idx(1, 1)
        wait_idx(0, 0)
        start_gather(0, 0)
        for i in range(zr // C):
            pltpu.sync_copy(msg_v.at[1],
                            agg_sh.at[pl.ds(sid * zr + i * C, C)])
        plsc.subcore_barrier()


        @pl.loop(0, nchunks, step=RING_I)
        def _(k0):
            for b in range(RING_I):
                kk = k0 + b
                m = b % 2
                bn = (b + 1) % RING_I
                bnn = (b + 2) % RING_I
                bp = (b + 3) % RING_I

                @pl.when(kk >= 1)
                def _():
                    wait_scatter(bp, 1 - m)

                @pl.when(kk < nchunks - 1)
                def _():
                    wait_idx(kk + 1, bn)
                    start_gather(bn, 1 - m)

                @pl.when(kk < nchunks - 2)
                def _():
                    start_idx(kk + 2, bnn)

                wait_gather(b, m)

                @plsc.parallel_loop(0, C // L, unroll=2)
                def _(g):
                    d16 = data_v[b, pl.ds(g * L, L)]
                    for e in range(L):
                        d = d16[e]
                        r = g * L + e
                        for j in range(H // L):
                            sl = pl.ds(j * L, L)
                            msg_v[m, r, sl] = msg_v[m, r, sl] * d

                start_scatter(b, m)

        wait_scatter((nchunks - 1) % RING_I, (nchunks - 1) % 2)
        plsc.subcore_barrier()

        pltpu.sync_copy(agg_sh.at[pl.ds(sid * zr, zr)],
                        out_hbm.at[cid, pl.ds(sid * zr, zr)])

    return sc_agg, np_pad


def kernel(node_features, adj_data, adj_row, adj_col, W_in, b_in,
           ln_in_scale, ln_in_offset, W_gc, b_gc, ln_fp_scale, ln_fp_offset,
           W_out, b_out):
    n = node_features.shape[0]
    e_total = adj_data.shape[0]
    grid = (n // BLK,)

    gran = RING_I * C
    ew = -(-e_total // (NW * gran)) * gran
    e_pad = NW * ew - e_total
    rows_per_tile = ew // C

    def pad2(a):
        return jnp.concatenate(
            [a, jnp.zeros((e_pad,), a.dtype)]).reshape(-1, C)

    col2 = pad2(adj_col)
    row2 = pad2(adj_row)
    data2 = pad2(adj_data)

    x, z, h = pl.pallas_call(
        _input_body,
        grid=grid,
        in_specs=[_row_spec(), _full_spec(H, H), _vec_spec(H), _vec_spec(H),
                  _vec_spec(H), _vec_spec(H), _vec_spec(H), _vec_spec(H),
                  _full_spec(H, H)],
        out_specs=[_row_spec(), _row_spec(), _row_spec()],
        out_shape=[jax.ShapeDtypeStruct((n, H), jnp.float32)] * 3,
    )(node_features, W_in, b_in, ln_in_scale, ln_in_offset, b_gc,
      ln_fp_scale, ln_fp_offset, W_gc)

    sc_agg, np_pad = _make_sc_agg(n, rows_per_tile)

    epi = pl.pallas_call(
        _epi_body,
        grid=grid,
        in_specs=[_row_spec(), _row_spec(), _row_spec(), _vec_spec(H),
                  _vec_spec(H), _vec_spec(H), _full_spec(H, H)],
        out_specs=[_row_spec(), _row_spec()],
        out_shape=[jax.ShapeDtypeStruct((n, H), jnp.float32)] * 2,
    )

    def body(carry, _):
        z, h = carry
        agg = sc_agg(h, col2, row2, data2)
        z, h = epi(agg[0], agg[1], x, b_gc, ln_fp_scale, ln_fp_offset, W_gc)
        return (z, h), None

    (z, h), _ = lax.scan(body, (z, h), None, length=MAXITER - 1)

    nc_out = W_out.shape[1]
    preds = pl.pallas_call(
        _out_body,
        grid=grid,
        in_specs=[_row_spec(), _full_spec(H, nc_out), _vec_spec(nc_out)],
        out_specs=pl.BlockSpec((BLK, nc_out), lambda i: (i, 0)),
        out_shape=jax.ShapeDtypeStruct((n, nc_out), jnp.float32),
    )(z, W_out, b_out)
    return preds
